# Initial kernel scaffold; baseline (speedup 1.0000x reference)
import jax
import jax.numpy as jnp
from jax.experimental import pallas as pl


def kernel(x, edge_index, W1, b1, W2, b2, group):
    raise NotImplementedError("write your pallas kernel here")


# speedup vs baseline: 9.7020x; 9.7020x over previous
import functools

import jax
import jax.numpy as jnp
from jax import lax
from jax.experimental import pallas as pl
from jax.experimental.pallas import tpu as pltpu
from jax.experimental.pallas import tpu_sc as plsc

_NC = 2
_NS = 16
_NW = _NC * _NS
_CH = 128


def _rup(a, b):
    return (a + b - 1) // b * b


@functools.lru_cache(maxsize=None)
def _make_deg_kernel(NP, NCH):
    RT = NP // _NS
    mesh = plsc.VectorSubcoreMesh(core_axis_name="c", subcore_axis_name="s")

    @functools.partial(
        pl.kernel,
        out_type=jax.ShapeDtypeStruct((_NC, NP, 16), jnp.float32),
        mesh=mesh,
        scratch_types=[
            pltpu.VMEM((_CH,), jnp.int32),
            pltpu.VMEM((_CH, 16), jnp.float32),
            pltpu.VMEM_SHARED((NP, 16), jnp.float32),
        ],
    )
    def kdeg(dstr_hbm, ones_hbm, zrow_hbm, out_hbm, dst_v, ones_v, acc_sh):
        c = lax.axis_index("c")
        s = lax.axis_index("s")
        wid = s * _NC + c
        pltpu.sync_copy(ones_hbm, ones_v)
        pltpu.sync_copy(zrow_hbm, acc_sh.at[pl.ds(s * RT, RT)])
        plsc.subcore_barrier()

        def body(j, carry):
            pltpu.sync_copy(dstr_hbm.at[wid, j], dst_v)
            pltpu.sync_copy(ones_v, acc_sh.at[dst_v], add=True)
            return carry

        lax.fori_loop(0, NCH, body, 0)
        plsc.subcore_barrier()
        pltpu.sync_copy(acc_sh.at[pl.ds(s * RT, RT)],
                        out_hbm.at[c, pl.ds(s * RT, RT)])

    return kdeg


@functools.lru_cache(maxsize=None)
def _make_edge_scatter_kernel(NP, Hd, NCH):
    RT = NP // _NS
    mesh = plsc.VectorSubcoreMesh(core_axis_name="c", subcore_axis_name="s")

    @functools.partial(
        pl.kernel,
        out_type=jax.ShapeDtypeStruct((_NC, NP, Hd), jnp.float32),
        mesh=mesh,
        scratch_types=[
            pltpu.VMEM((_CH,), jnp.int32),
            pltpu.VMEM((_CH,), jnp.int32),
            pltpu.VMEM((_CH, Hd), jnp.float32),
            pltpu.VMEM_SHARED((NP, Hd), jnp.float32),
            pltpu.SemaphoreType.DMA,
        ],
    )
    def kscat(hs_hbm, srcr_hbm, dstr_hbm, zrow_hbm, out_hbm,
              src_v, dst_v, rows_v, acc_sh, sem):
        c = lax.axis_index("c")
        s = lax.axis_index("s")
        wid = s * _NC + c
        pltpu.sync_copy(zrow_hbm, acc_sh.at[pl.ds(s * RT, RT)])
        plsc.subcore_barrier()

        def body(j, carry):
            pltpu.sync_copy(srcr_hbm.at[wid, j], src_v)
            pltpu.sync_copy(dstr_hbm.at[wid, j], dst_v)
            pltpu.async_copy(hs_hbm.at[src_v], rows_v, sem).wait()
            pltpu.sync_copy(rows_v, acc_sh.at[dst_v], add=True)
            return carry

        lax.fori_loop(0, NCH, body, 0)
        plsc.subcore_barrier()
        pltpu.sync_copy(acc_sh.at[pl.ds(s * RT, RT)],
                        out_hbm.at[c, pl.ds(s * RT, RT)])

    return kscat


def _prep_body(x_ref, w1_ref, degp_ref, hs_ref, dinv_ref):
    x = x_ref[...]
    rowsum = jnp.sum(x, axis=1, keepdims=True)
    rinv = jnp.where(rowsum != 0.0, 1.0 / rowsum, 0.0)
    xn = x * rinv
    deg = degp_ref[0, :, 0:1] + degp_ref[1, :, 0:1] + 1.0
    dinv = lax.rsqrt(deg)
    h = jnp.dot(xn, w1_ref[...], preferred_element_type=jnp.float32)
    hs_ref[...] = h * dinv
    dinv_ref[...] = dinv


def _mid_body(accp_ref, hs_ref, dinv_ref, b_ref, w2_ref, hs2_ref):
    dinv = dinv_ref[...]
    out1 = (accp_ref[0] + accp_ref[1] + hs_ref[...]) * dinv + b_ref[...]
    h1 = jnp.maximum(out1, 0.0)
    h2 = jnp.dot(h1, w2_ref[...], preferred_element_type=jnp.float32)
    hs2_ref[...] = h2 * dinv


def _fin_body(accp_ref, hs_ref, dinv_ref, b_ref, out_ref):
    out_ref[...] = ((accp_ref[0] + accp_ref[1] + hs_ref[...]) * dinv_ref[...]
                    + b_ref[...])


def kernel(x, edge_index, W1, b1, W2, b2, group):
    N, D = x.shape
    H = W1.shape[1]
    E = edge_index.shape[1]

    NP = _rup(N + 1, _NS * 8)
    EW = _rup(-(-E // _NW), _CH)
    NCH = EW // _CH

    pad_e = EW * _NW - E
    src = jnp.concatenate(
        [edge_index[0], jnp.full((pad_e,), N, jnp.int32)]).reshape(_NW, NCH, _CH)
    dst = jnp.concatenate(
        [edge_index[1], jnp.full((pad_e,), N, jnp.int32)]).reshape(_NW, NCH, _CH)

    xpad = jnp.zeros((NP, D), jnp.float32).at[:N].set(x)
    RT = NP // _NS
    zrow16 = jnp.zeros((RT, 16), jnp.float32)
    zrowH = jnp.zeros((RT, H), jnp.float32)
    ones16 = jnp.ones((_CH, 16), jnp.float32)
    b1r = b1.reshape(1, H)
    b2r = b2.reshape(1, H)

    degp = _make_deg_kernel(NP, NCH)(dst, ones16, zrow16)

    hs1, dinv = pl.pallas_call(
        _prep_body,
        out_shape=(
            jax.ShapeDtypeStruct((NP, H), jnp.float32),
            jax.ShapeDtypeStruct((NP, 1), jnp.float32),
        ),
    )(xpad, W1, degp)

    edge_scatter = _make_edge_scatter_kernel(NP, H, NCH)
    acc1 = edge_scatter(hs1, src, dst, zrowH)

    hs2 = pl.pallas_call(
        _mid_body,
        out_shape=jax.ShapeDtypeStruct((NP, H), jnp.float32),
    )(acc1, hs1, dinv, b1r, W2)

    acc2 = edge_scatter(hs2, src, dst, zrowH)

    out = pl.pallas_call(
        _fin_body,
        out_shape=jax.ShapeDtypeStruct((NP, H), jnp.float32),
    )(acc2, hs2, dinv, b2r)

    return out[:N]
